# Initial kernel scaffold; baseline (speedup 1.0000x reference)
import jax
import jax.numpy as jnp
from jax.experimental import pallas as pl


def kernel(x, weight, bias):
    raise NotImplementedError("write your pallas kernel here")


# speedup vs baseline: 5.3790x; 5.3790x over previous
import jax
import jax.numpy as jnp
from jax import lax
from jax.experimental import pallas as pl
from jax.experimental.pallas import tpu as pltpu

_MIB = 1 << 20


def _linear_bf16_kernel(x_ref, w_ref, b_ref, o_ref):
    xb = x_ref[...].astype(jnp.bfloat16)
    acc = lax.dot_general(
        xb,
        w_ref[...],
        dimension_numbers=(((1,), (1,)), ((), ())),
        preferred_element_type=jnp.float32,
    )
    o_ref[...] = acc + b_ref[...]


def kernel(x, weight, bias):
    B, in_size = x.shape
    out_size = weight.shape[0]

    w_bf16 = weight.astype(jnp.bfloat16)
    b2 = bias.reshape(1, out_size)

    tm = min(256, B)
    grid = (pl.cdiv(B, tm),)

    working = (
        out_size * in_size * 2
        + 2 * tm * in_size * 4
        + 2 * tm * out_size * 4
        + tm * in_size * 2
        + out_size * 4
    )
    return pl.pallas_call(
        _linear_bf16_kernel,
        out_shape=jax.ShapeDtypeStruct((B, out_size), x.dtype),
        grid_spec=pl.GridSpec(
            grid=grid,
            in_specs=[
                pl.BlockSpec((tm, in_size), lambda i: (i, 0)),
                pl.BlockSpec((out_size, in_size), lambda i: (0, 0),
                             pipeline_mode=pl.Buffered(1)),
                pl.BlockSpec((1, out_size), lambda i: (0, 0),
                             pipeline_mode=pl.Buffered(1)),
            ],
            out_specs=pl.BlockSpec((tm, out_size), lambda i: (i, 0)),
        ),
        compiler_params=pltpu.CompilerParams(
            dimension_semantics=("parallel",),
            vmem_limit_bytes=int(min(working + 8 * _MIB, 62 * _MIB)),
        ),
        cost_estimate=pl.CostEstimate(
            flops=2 * B * in_size * out_size,
            transcendentals=0,
            bytes_accessed=4 * (B * in_size + B * out_size + out_size)
            + 2 * out_size * in_size,
        ),
    )(x, w_bf16, b2)


# speedup vs baseline: 5.7115x; 1.0618x over previous
import jax
import jax.numpy as jnp
from jax import lax
from jax.experimental import pallas as pl
from jax.experimental.pallas import tpu as pltpu

_MIB = 1 << 20


def _fused_kernel(x_ref, w_ref, b_ref, o_ref, wb_ref):
    i = pl.program_id(1)

    @pl.when(i == 0)
    def _():
        wb_ref[...] = w_ref[...].astype(jnp.bfloat16)

    acc = lax.dot_general(
        x_ref[...].astype(jnp.bfloat16),
        wb_ref[...],
        dimension_numbers=(((1,), (1,)), ((), ())),
        preferred_element_type=jnp.float32,
    )
    o_ref[...] = acc + b_ref[...]


def kernel(x, weight, bias):
    B, in_size = x.shape
    out_size = weight.shape[0]
    b2 = bias.reshape(1, out_size)

    tn = out_size // 2
    tm = min(256, B)
    grid = (2, pl.cdiv(B, tm))

    working = (
        tn * in_size * 4
        + tn * in_size * 2
        + 2 * tm * in_size * 4
        + 2 * tm * tn * 4
        + out_size * 4
    )
    return pl.pallas_call(
        _fused_kernel,
        out_shape=jax.ShapeDtypeStruct((B, out_size), jnp.float32),
        grid_spec=pltpu.PrefetchScalarGridSpec(
            num_scalar_prefetch=0,
            grid=grid,
            in_specs=[
                pl.BlockSpec((tm, in_size), lambda j, i: (i, 0)),
                pl.BlockSpec((tn, in_size), lambda j, i: (j, 0),
                             pipeline_mode=pl.Buffered(1)),
                pl.BlockSpec((1, tn), lambda j, i: (0, j),
                             pipeline_mode=pl.Buffered(1)),
            ],
            out_specs=pl.BlockSpec((tm, tn), lambda j, i: (i, j)),
            scratch_shapes=[pltpu.VMEM((tn, in_size), jnp.bfloat16)],
        ),
        compiler_params=pltpu.CompilerParams(
            dimension_semantics=("parallel", "arbitrary"),
            vmem_limit_bytes=int(min(working + 6 * _MIB, 62 * _MIB)),
        ),
        cost_estimate=pl.CostEstimate(
            flops=2 * B * in_size * out_size,
            transcendentals=0,
            bytes_accessed=4 * (2 * B * in_size + out_size * in_size
                                + B * out_size + out_size),
        ),
    )(x, weight, b2)
